# baseline (speedup 1.0000x reference)
import functools

import jax
import jax.numpy as jnp
from jax import lax
from jax.experimental import pallas as pl
from jax.experimental.pallas import tpu as pltpu
from jax.experimental.pallas import tpu_sc as plsc

N = 10000
E = 320000
H = 128
HC = 64
NPAD = 10240
NC = 2
NS = 16
CH = 128
NB = 4
NCH = NB * (-(-E // (NS * CH * NB)))
EPW = NCH * CH
EPAD = NS * EPW
RPS = NPAD // NS
NZCH = RPS // CH

_mesh = plsc.VectorSubcoreMesh(core_axis_name="c", subcore_axis_name="s")


@functools.partial(
    pl.kernel,
    out_type=[jax.ShapeDtypeStruct((NC, NPAD, HC), jnp.float32),
              jax.ShapeDtypeStruct((NC, NPAD), jnp.float32)],
    mesh=_mesh,
    scratch_types=[
        pltpu.VMEM((NCH, CH), jnp.int32),
        pltpu.VMEM((NCH, CH), jnp.int32),
        pltpu.VMEM((NB, CH, HC), jnp.float32),
        pltpu.VMEM((CH,), jnp.float32),
        pltpu.VMEM((RPS,), jnp.float32),
        pltpu.VMEM_SHARED((NPAD, HC), jnp.float32),
        pltpu.VMEM_SHARED((NPAD,), jnp.float32),
        pltpu.SemaphoreType.DMA((NB,)),
        pltpu.SemaphoreType.DMA,
    ],
    compiler_params=pltpu.CompilerParams(use_tc_tiling_on_sc=False),
)
def _sc_agg(h2_hbm, srcl, dstl, z_hbm, zn_hbm, ones_hbm, parts, degp,
            src_v, dst_v, bufs, ones_v, stage_v, acc, dacc, sems, sem_d):
    c = lax.axis_index("c")
    s = lax.axis_index("s")
    h_hbm = h2_hbm.at[c]
    pltpu.sync_copy(z_hbm, bufs.at[0])
    for k in range(NZCH):
        pltpu.sync_copy(bufs.at[0], acc.at[pl.ds((s * NZCH + k) * CH, CH)])
    pltpu.sync_copy(zn_hbm.at[pl.ds(s * RPS, RPS)], stage_v)
    pltpu.sync_copy(stage_v, dacc.at[pl.ds(s * RPS, RPS)])
    pltpu.sync_copy(ones_hbm, ones_v)
    pltpu.sync_copy(srcl.at[s], src_v)
    pltpu.sync_copy(dstl.at[s], dst_v)
    plsc.subcore_barrier()

    def _gwait(b):
        pltpu.make_async_copy(
            h_hbm.at[pl.ds(0, CH)], bufs.at[b], sems.at[b]).wait()

    for b in range(NB):
        pltpu.async_copy(h_hbm.at[src_v.at[b]], bufs.at[b], sems.at[b])

    def step(t, carry):
        j = NB * t
        for b in range(NB):
            _gwait(b)
            pltpu.sync_copy(bufs.at[b], acc.at[dst_v.at[j + b]], add=True)
            pltpu.async_copy(ones_v, dacc.at[dst_v.at[j + b]], sem_d,
                             add=True)
            pltpu.async_copy(h_hbm.at[src_v.at[j + b + NB]], bufs.at[b],
                             sems.at[b])
        return carry

    lax.fori_loop(0, NCH // NB - 1, step, 0)
    for b in range(NB):
        _gwait(b)
        pltpu.sync_copy(bufs.at[b], acc.at[dst_v.at[NCH - NB + b]], add=True)
        pltpu.async_copy(ones_v, dacc.at[dst_v.at[NCH - NB + b]], sem_d,
                         add=True)
    pltpu.make_async_copy(srcl.at[s], src_v, sem_d).wait()
    plsc.subcore_barrier()
    for k in range(NZCH):
        base = (s * NZCH + k) * CH
        pltpu.sync_copy(acc.at[pl.ds(base, CH)], bufs.at[0])
        pltpu.sync_copy(bufs.at[0], parts.at[c, pl.ds(base, CH)])
    pltpu.sync_copy(dacc.at[pl.ds(s * RPS, RPS)], stage_v)
    pltpu.sync_copy(stage_v, degp.at[c, pl.ds(s * RPS, RPS)])


BLK = 10240


def _tc_layer_body(parts_ref, degp_ref, h2_ref, wl_ref, wr_ref, b_ref,
                   out_ref, hrelu2_ref):
    agg = jnp.concatenate([parts_ref[0], parts_ref[1]], axis=-1)
    hh = jnp.concatenate([h2_ref[0], h2_ref[1]], axis=-1)
    deg = degp_ref[0]
    inv = 1.0 / jnp.maximum(deg, 1.0)
    mean = agg * inv[:, None]
    val = (jnp.dot(mean, wl_ref[...], preferred_element_type=jnp.float32)
           + jnp.dot(hh, wr_ref[...], preferred_element_type=jnp.float32)
           + b_ref[...])
    out_ref[...] = val
    if hrelu2_ref is not None:
        hr = jnp.maximum(val, 0.0)
        hrelu2_ref[0] = hr[:, :HC]
        hrelu2_ref[1] = hr[:, HC:]


def _tc_layer(parts, degp, h2, Wl, Wr, b, want_relu=True):
    nb = NPAD // BLK
    out_specs = [pl.BlockSpec((BLK, H), lambda i: (i, 0))]
    out_shape = [jax.ShapeDtypeStruct((NPAD, H), jnp.float32)]
    body = _tc_layer_body
    if want_relu:
        out_specs.append(pl.BlockSpec((NC, BLK, HC), lambda i: (0, i, 0)))
        out_shape.append(jax.ShapeDtypeStruct((NC, NPAD, HC), jnp.float32))
    else:
        body = functools.partial(_tc_layer_body, hrelu2_ref=None)
    res = pl.pallas_call(
        body,
        grid=(nb,),
        in_specs=[
            pl.BlockSpec((NC, BLK, HC), lambda i: (0, i, 0)),
            pl.BlockSpec((NC, BLK), lambda i: (0, i)),
            pl.BlockSpec((NC, BLK, HC), lambda i: (0, i, 0)),
            pl.BlockSpec((H, H), lambda i: (0, 0)),
            pl.BlockSpec((H, H), lambda i: (0, 0)),
            pl.BlockSpec((1, H), lambda i: (0, 0)),
        ],
        out_specs=out_specs,
        out_shape=out_shape,
    )(parts, degp, h2, Wl, Wr, b)
    return res if want_relu else (res[0], None)


def kernel(x, edge_index, W0l, W0r, b0, W1l, W1r, b1, W2l, W2r, b2):
    f32 = jnp.float32
    xpad = jnp.zeros((NPAD, H), f32).at[:N].set(x)
    h2 = jnp.stack([xpad[:, :HC], xpad[:, HC:]])
    padn = EPAD - E
    ar = jnp.arange(padn, dtype=jnp.int32)
    src = jnp.concatenate([edge_index[0], ar % N])
    dst = jnp.concatenate([edge_index[1], N + ar % (NPAD - N)])
    srcl = src.reshape(NS, NCH, CH)
    dstl = dst.reshape(NS, NCH, CH)
    z64 = jnp.zeros((CH, HC), f32)
    zn = jnp.zeros((NPAD,), f32)
    ones = jnp.ones((CH,), f32)
    outs = []
    layers = ((W0l, W0r, b0), (W1l, W1r, b1), (W2l, W2r, b2))
    for li, (Wl, Wr, b) in enumerate(layers):
        parts, degp = _sc_agg(h2, srcl, dstl, z64, zn, ones)
        out, h2 = _tc_layer(parts, degp, h2, Wl, Wr, b.reshape(1, H),
                            want_relu=li < 2)
        outs.append(out[:N])
    return jnp.stack(outs, axis=1)


# speedup vs baseline: 1.0701x; 1.0701x over previous
import functools

import jax
import jax.numpy as jnp
from jax import lax
from jax.experimental import pallas as pl
from jax.experimental.pallas import tpu as pltpu
from jax.experimental.pallas import tpu_sc as plsc

N = 10000
E = 320000
H = 128
HC = 64
NPAD = 10240
NC = 2
NS = 16
CH = 128
NB = 4
NCH = NB * (-(-E // (NS * CH * NB)))
EPW = NCH * CH
EPAD = NS * EPW
RPS = NPAD // NS
NZCH = RPS // CH

_mesh = plsc.VectorSubcoreMesh(core_axis_name="c", subcore_axis_name="s")


def _make_sc_agg(with_deg):
    out_type = [jax.ShapeDtypeStruct((NC, NPAD, HC), jnp.float32)]
    scratch = [
        pltpu.VMEM((NCH, CH), jnp.int32),
        pltpu.VMEM((NCH, CH), jnp.int32),
        pltpu.VMEM((NB, CH, HC), jnp.float32),
        pltpu.VMEM_SHARED((NPAD, HC), jnp.float32),
        pltpu.SemaphoreType.DMA((NB,)),
    ]
    if with_deg:
        out_type.append(jax.ShapeDtypeStruct((NC, NPAD), jnp.float32))
        scratch += [
            pltpu.VMEM((CH,), jnp.float32),
            pltpu.VMEM((RPS,), jnp.float32),
            pltpu.VMEM_SHARED((NPAD,), jnp.float32),
            pltpu.SemaphoreType.DMA,
        ]

    @functools.partial(
        pl.kernel,
        out_type=out_type,
        mesh=_mesh,
        scratch_types=scratch,
        compiler_params=pltpu.CompilerParams(use_tc_tiling_on_sc=False),
    )
    def _body(h2_hbm, srcl, dstl, z_hbm, zn_hbm, ones_hbm, *rest):
        if with_deg:
            (parts, degp, src_v, dst_v, bufs, acc, sems,
             ones_v, stage_v, dacc, sem_d) = rest
        else:
            parts, src_v, dst_v, bufs, acc, sems = rest
        c = lax.axis_index("c")
        s = lax.axis_index("s")
        h_hbm = h2_hbm.at[c]
        pltpu.sync_copy(z_hbm, bufs.at[0])
        for k in range(NZCH):
            pltpu.sync_copy(bufs.at[0], acc.at[pl.ds((s * NZCH + k) * CH, CH)])
        if with_deg:
            pltpu.sync_copy(zn_hbm.at[pl.ds(s * RPS, RPS)], stage_v)
            pltpu.sync_copy(stage_v, dacc.at[pl.ds(s * RPS, RPS)])
            pltpu.sync_copy(ones_hbm, ones_v)
        pltpu.sync_copy(srcl.at[s], src_v)
        pltpu.sync_copy(dstl.at[s], dst_v)
        plsc.subcore_barrier()

        def _gwait(b):
            pltpu.make_async_copy(
                h_hbm.at[pl.ds(0, CH)], bufs.at[b], sems.at[b]).wait()

        def _chunk(j, b):
            _gwait(b)
            pltpu.sync_copy(bufs.at[b], acc.at[dst_v.at[j]], add=True)
            if with_deg:
                pltpu.async_copy(ones_v, dacc.at[dst_v.at[j]], sem_d,
                                 add=True)

        for b in range(NB):
            pltpu.async_copy(h_hbm.at[src_v.at[b]], bufs.at[b], sems.at[b])

        def step(t, carry):
            j = NB * t
            for b in range(NB):
                _chunk(j + b, b)
                pltpu.async_copy(h_hbm.at[src_v.at[j + b + NB]], bufs.at[b],
                                 sems.at[b])
            return carry

        lax.fori_loop(0, NCH // NB - 1, step, 0)
        for b in range(NB):
            _chunk(NCH - NB + b, b)
        if with_deg:
            pltpu.make_async_copy(srcl.at[s], src_v, sem_d).wait()
        plsc.subcore_barrier()
        for k in range(NZCH):
            base = (s * NZCH + k) * CH
            pltpu.sync_copy(acc.at[pl.ds(base, CH)], bufs.at[0])
            pltpu.sync_copy(bufs.at[0], parts.at[c, pl.ds(base, CH)])
        if with_deg:
            pltpu.sync_copy(dacc.at[pl.ds(s * RPS, RPS)], stage_v)
            pltpu.sync_copy(stage_v, degp.at[c, pl.ds(s * RPS, RPS)])

    return _body


_sc_agg_deg = _make_sc_agg(True)
_sc_agg_nodeg = _make_sc_agg(False)


BLK = 5120


def _tc_layer_body(parts_ref, degp_ref, h2_ref, wl_ref, wr_ref, b_ref,
                   out_ref, hrelu2_ref):
    agg = jnp.concatenate([parts_ref[0], parts_ref[1]], axis=-1)
    hh = jnp.concatenate([h2_ref[0], h2_ref[1]], axis=-1)
    deg = degp_ref[0]
    inv = 1.0 / jnp.maximum(deg, 1.0)
    mean = agg * inv[:, None]
    val = (jnp.dot(mean, wl_ref[...], preferred_element_type=jnp.float32)
           + jnp.dot(hh, wr_ref[...], preferred_element_type=jnp.float32)
           + b_ref[...])
    out_ref[...] = val
    if hrelu2_ref is not None:
        hr = jnp.maximum(val, 0.0)
        hrelu2_ref[0] = hr[:, :HC]
        hrelu2_ref[1] = hr[:, HC:]


def _tc_layer(parts, degp, h2, Wl, Wr, b, want_relu=True):
    nb = NPAD // BLK
    out_specs = [pl.BlockSpec((BLK, H), lambda i: (i, 0))]
    out_shape = [jax.ShapeDtypeStruct((NPAD, H), jnp.float32)]
    body = _tc_layer_body
    if want_relu:
        out_specs.append(pl.BlockSpec((NC, BLK, HC), lambda i: (0, i, 0)))
        out_shape.append(jax.ShapeDtypeStruct((NC, NPAD, HC), jnp.float32))
    else:
        body = functools.partial(_tc_layer_body, hrelu2_ref=None)
    res = pl.pallas_call(
        body,
        grid=(nb,),
        in_specs=[
            pl.BlockSpec((NC, BLK, HC), lambda i: (0, i, 0)),
            pl.BlockSpec((NC, BLK), lambda i: (0, i)),
            pl.BlockSpec((NC, BLK, HC), lambda i: (0, i, 0)),
            pl.BlockSpec((H, H), lambda i: (0, 0)),
            pl.BlockSpec((H, H), lambda i: (0, 0)),
            pl.BlockSpec((1, H), lambda i: (0, 0)),
        ],
        out_specs=out_specs,
        out_shape=out_shape,
    )(parts, degp, h2, Wl, Wr, b)
    return res if want_relu else (res[0], None)


def kernel(x, edge_index, W0l, W0r, b0, W1l, W1r, b1, W2l, W2r, b2):
    f32 = jnp.float32
    xpad = jnp.zeros((NPAD, H), f32).at[:N].set(x)
    h2 = jnp.stack([xpad[:, :HC], xpad[:, HC:]])
    padn = EPAD - E
    ar = jnp.arange(padn, dtype=jnp.int32)
    src = jnp.concatenate([edge_index[0], ar % N])
    dst = jnp.concatenate([edge_index[1], N + ar % (NPAD - N)])
    srcl = src.reshape(NS, NCH, CH)
    dstl = dst.reshape(NS, NCH, CH)
    z64 = jnp.zeros((CH, HC), f32)
    zn = jnp.zeros((NPAD,), f32)
    ones = jnp.ones((CH,), f32)
    outs = []
    degp = None
    layers = ((W0l, W0r, b0), (W1l, W1r, b1), (W2l, W2r, b2))
    for li, (Wl, Wr, b) in enumerate(layers):
        if li == 0:
            parts, degp = _sc_agg_deg(h2, srcl, dstl, z64, zn, ones)
        else:
            (parts,) = _sc_agg_nodeg(h2, srcl, dstl, z64, zn, ones)
        out, h2 = _tc_layer(parts, degp, h2, Wl, Wr, b.reshape(1, H),
                            want_relu=li < 2)
        outs.append(out[:N])
    return jnp.stack(outs, axis=1)
